# baseline (speedup 1.0000x reference)
import functools

import jax
import jax.numpy as jnp
import numpy as np
from jax import lax
from jax.experimental import pallas as pl
from jax.experimental.pallas import tpu as pltpu
from jax.experimental.pallas import tpu_sc as plsc

N_NODES = 10000
NPAD = 10112
ROWS_PER_SUB = NPAD // 16
N_EDGES = 320000
NW = 32
CHUNK = 128
NCH = 80
NE_PAD = NW * NCH * CHUNK
RSW = 16
BATCH = 4096
NBUF = 8

_mesh = plsc.VectorSubcoreMesh(core_axis_name="c", subcore_axis_name="s")
_sc_params = pltpu.CompilerParams(use_tc_tiling_on_sc=False)


@functools.partial(
    pl.kernel,
    out_type=[
        jax.ShapeDtypeStruct((NPAD, RSW), jnp.float32),
        jax.ShapeDtypeStruct((NPAD, RSW), jnp.float32),
    ],
    mesh=_mesh,
    compiler_params=_sc_params,
    scratch_types=[
        pltpu.VMEM((NCH, CHUNK), jnp.int32),
        pltpu.VMEM((CHUNK, RSW), jnp.float32),
        pltpu.VMEM_SHARED((NPAD, RSW), jnp.float32),
        [pltpu.SemaphoreType.DMA for _ in range(8)],
    ],
)
def _deg_kernel(dst_hbm, ones_hbm, zeros_hbm, out0, out1, idx_v, ones_v, acc,
                sems):
    c = lax.axis_index("c")
    s = lax.axis_index("s")
    w = c * 16 + s
    pltpu.sync_copy(zeros_hbm, acc.at[pl.ds(s * ROWS_PER_SUB, ROWS_PER_SUB)])
    pltpu.sync_copy(ones_hbm, ones_v)
    pltpu.sync_copy(dst_hbm.at[w], idx_v)
    plsc.subcore_barrier()

    def step(j, carry):
        c0 = j * 8
        for k in range(8):
            pltpu.async_copy(ones_v, acc.at[idx_v.at[c0 + k]], sems[k], add=True)
        for k in range(8):
            pltpu.make_async_copy(ones_v, acc.at[idx_v.at[0]], sems[k]).wait()
        return carry

    lax.fori_loop(0, NCH // 8, step, 0)
    plsc.subcore_barrier()
    r0 = s * ROWS_PER_SUB
    sl = pl.ds(r0, ROWS_PER_SUB)

    @pl.when(c == 0)
    def _():
        pltpu.sync_copy(acc.at[sl], out0.at[sl])

    @pl.when(c == 1)
    def _():
        pltpu.sync_copy(acc.at[sl], out1.at[sl])


def _make_agg_kernel(feat):
    @functools.partial(
        pl.kernel,
        out_type=[
            jax.ShapeDtypeStruct((NPAD, feat), jnp.float32),
            jax.ShapeDtypeStruct((NPAD, feat), jnp.float32),
        ],
        mesh=_mesh,
        compiler_params=_sc_params,
        scratch_types=[
            pltpu.VMEM((NCH, CHUNK), jnp.int32),
            pltpu.VMEM((NCH, CHUNK), jnp.int32),
            [pltpu.VMEM((CHUNK, feat), jnp.float32) for _ in range(NBUF)],
            pltpu.VMEM_SHARED((NPAD, feat), jnp.float32),
            [pltpu.SemaphoreType.DMA for _ in range(NBUF)],
            [pltpu.SemaphoreType.DMA for _ in range(NBUF)],
        ],
    )
    def _agg(src_hbm, dst_hbm, g_hbm, zeros_hbm, out0, out1,
             srcv, dstv, rows, acc, gsem, ssem):
        c = lax.axis_index("c")
        s = lax.axis_index("s")
        w = c * 16 + s
        pltpu.sync_copy(zeros_hbm, acc.at[pl.ds(s * ROWS_PER_SUB, ROWS_PER_SUB)])
        pltpu.sync_copy(src_hbm.at[w], srcv)
        pltpu.sync_copy(dst_hbm.at[w], dstv)
        plsc.subcore_barrier()

        for k in range(NBUF):
            pltpu.async_copy(g_hbm.at[srcv.at[k]], rows[k], gsem[k])

        def step(j, carry):
            c0 = j * NBUF
            for k in range(NBUF):
                pltpu.make_async_copy(g_hbm.at[srcv.at[0]], rows[k], gsem[k]).wait()
                pltpu.async_copy(rows[k], acc.at[dstv.at[c0 + k]], ssem[k],
                                 add=True)
            for k in range(NBUF):
                pltpu.make_async_copy(rows[k], acc.at[dstv.at[0]], ssem[k]).wait()
                pltpu.async_copy(g_hbm.at[srcv.at[c0 + NBUF + k]], rows[k],
                                 gsem[k])
            return carry

        lax.fori_loop(0, NCH // NBUF - 1, step, 0)
        c0 = NCH - NBUF
        for k in range(NBUF):
            pltpu.make_async_copy(g_hbm.at[srcv.at[0]], rows[k], gsem[k]).wait()
            pltpu.async_copy(rows[k], acc.at[dstv.at[c0 + k]], ssem[k], add=True)
        for k in range(NBUF):
            pltpu.make_async_copy(rows[k], acc.at[dstv.at[0]], ssem[k]).wait()
        plsc.subcore_barrier()
        sl = pl.ds(s * ROWS_PER_SUB, ROWS_PER_SUB)

        @pl.when(c == 0)
        def _():
            pltpu.sync_copy(acc.at[sl], out0.at[sl])

        @pl.when(c == 1)
        def _():
            pltpu.sync_copy(acc.at[sl], out1.at[sl])

    return _agg


_agg64 = _make_agg_kernel(64)
_agg32 = _make_agg_kernel(32)


PBATCH = 2 * BATCH
PW = PBATCH // NW
PCH = PW // CHUNK


@functools.partial(
    pl.kernel,
    out_type=[
        jax.ShapeDtypeStruct((PBATCH, 32), jnp.float32),
        jax.ShapeDtypeStruct((PBATCH, RSW), jnp.float32),
    ],
    mesh=_mesh,
    compiler_params=_sc_params,
    scratch_types=[
        pltpu.VMEM((PCH, CHUNK), jnp.int32),
        pltpu.VMEM((CHUNK, 32), jnp.float32),
        pltpu.VMEM((CHUNK, 32), jnp.float32),
        pltpu.VMEM((CHUNK, 32), jnp.float32),
        pltpu.VMEM((CHUNK, RSW), jnp.float32),
        pltpu.SemaphoreType.DMA,
    ],
)
def _pair_gather(pc_hbm, t20_hbm, t21_hbm, g2_hbm, rs_hbm,
                 s_out, grs_out, idx2, b0, b1, b2, br, sem):
    w = _worker_id()
    pltpu.sync_copy(pc_hbm.at[w], idx2)
    for j in range(PCH):
        base = w * (PCH * CHUNK) + j * CHUNK
        pltpu.async_copy(t20_hbm.at[idx2.at[j]], b0, sem)
        pltpu.async_copy(t21_hbm.at[idx2.at[j]], b1, sem)
        pltpu.async_copy(g2_hbm.at[idx2.at[j]], b2, sem)
        pltpu.async_copy(rs_hbm.at[idx2.at[j]], br, sem)
        pltpu.make_async_copy(t20_hbm.at[idx2.at[j]], b0, sem).wait()
        pltpu.make_async_copy(t20_hbm.at[idx2.at[j]], b1, sem).wait()
        pltpu.make_async_copy(t20_hbm.at[idx2.at[j]], b2, sem).wait()
        pltpu.make_async_copy(rs_hbm.at[idx2.at[j]], br, sem).wait()

        def vsum(i, carry):
            r = i // 2
            l = (i % 2) * 16
            b0[r, pl.ds(l, 16)] = (b0[r, pl.ds(l, 16)] + b1[r, pl.ds(l, 16)]
                                   + b2[r, pl.ds(l, 16)])
            return carry

        lax.fori_loop(0, CHUNK * 2, vsum, 0)
        pltpu.sync_copy(b0, s_out.at[pl.ds(base, CHUNK)])
        pltpu.sync_copy(br, grs_out.at[pl.ds(base, CHUNK)])


def _worker_id():
    return lax.axis_index("c") * 16 + lax.axis_index("s")


_BLK = 1024
_GRID = (NPAD + _BLK - 1) // _BLK


def _mmh_body(x_ref, w_ref, h_ref):
    h_ref[...] = jnp.dot(x_ref[...], w_ref[...],
                         preferred_element_type=jnp.float32)


def _mmh(x, Wpe):
    return pl.pallas_call(
        _mmh_body,
        grid=(_GRID,),
        in_specs=[
            pl.BlockSpec((_BLK, 128), lambda i: (i, 0)),
            pl.BlockSpec((128, 64), lambda i: (0, 0)),
        ],
        out_specs=pl.BlockSpec((_BLK, 64), lambda i: (i, 0)),
        out_shape=jax.ShapeDtypeStruct((NPAD, 64), jnp.float32),
    )(x, Wpe)


def _mm2_body(h0_ref, w_ref, g2_ref):
    g2_ref[...] = jnp.dot(h0_ref[...], w_ref[...],
                          preferred_element_type=jnp.float32)


def _mm2(h0, Wse):
    return pl.pallas_call(
        _mm2_body,
        grid=(_GRID,),
        in_specs=[
            pl.BlockSpec((_BLK, 64), lambda i: (i, 0)),
            pl.BlockSpec((64, 32), lambda i: (0, 0)),
        ],
        out_specs=pl.BlockSpec((_BLK, 32), lambda i: (i, 0)),
        out_shape=jax.ShapeDtypeStruct((NPAD, 32), jnp.float32),
    )(h0, Wse)


def _softmax2(z):
    m = jnp.max(z, axis=-1, keepdims=True)
    e = jnp.exp(z - m)
    return e / jnp.sum(e, axis=-1, keepdims=True)


def _dec_body(hadd_ref, eps_ref,
              wm_ref, bm_ref, wls_ref, bls_ref, wsd_ref, bsd_ref,
              wpd1_ref, bpd1_ref, wpd2_ref, bpd2_ref,
              wa1_ref, ba1_ref, wa2_ref, ba2_ref,
              wd1_ref, bd1_ref, wd2_ref, bd2_ref,
              xrec_ref, ap_ref, dp_ref, mean_ref, logstd_ref):
    hadd = hadd_ref[...]

    def mm(a, w_ref, b_ref):
        return jnp.dot(a, w_ref[...], preferred_element_type=jnp.float32) + b_ref[...]

    mean = mm(hadd, wm_ref, bm_ref)
    logstd = mm(hadd, wls_ref, bls_ref)
    mean_ref[...] = mean
    logstd_ref[...] = logstd
    nz = mean + eps_ref[...] * jnp.exp(logstd)
    xr = jnp.maximum(mm(nz, wsd_ref, bsd_ref), 0.0)
    xr = jnp.maximum(mm(xr, wpd1_ref, bpd1_ref), 0.0)
    xrec_ref[...] = mm(xr, wpd2_ref, bpd2_ref)
    ap_ref[...] = _softmax2(mm(jnp.maximum(mm(nz, wa1_ref, ba1_ref), 0.0),
                               wa2_ref, ba2_ref))
    dp_ref[...] = _softmax2(mm(jnp.maximum(mm(nz, wd1_ref, bd1_ref), 0.0),
                               wd2_ref, bd2_ref))


def _decoder(hadd, eps, Wm, bm, Wls, bls, Wsd, bsd,
             Wpd1, bpd1, Wpd2, bpd2, Wa1, ba1, Wa2, ba2, Wd1, bd1, Wd2, bd2):
    return pl.pallas_call(
        _dec_body,
        out_shape=[
            jax.ShapeDtypeStruct((BATCH, 128), jnp.float32),
            jax.ShapeDtypeStruct((BATCH, 2), jnp.float32),
            jax.ShapeDtypeStruct((BATCH, 2), jnp.float32),
            jax.ShapeDtypeStruct((BATCH, 16), jnp.float32),
            jax.ShapeDtypeStruct((BATCH, 16), jnp.float32),
        ],
    )(hadd, eps, Wm, bm, Wls, bls, Wsd, bsd,
      Wpd1, bpd1, Wpd2, bpd2, Wa1, ba1, Wa2, ba2, Wd1, bd1, Wd2, bd2)


def kernel(x, edge_index, node_pair, rate, Wpe, bpe, Wse, bse, Wm, bm, Wls,
           bls, Wsd, bsd, Wpd1, bpd1, Wpd2, bpd2, Wa1, ba1, Wa2, ba2, Wd1,
           bd1, Wd2, bd2):
    f32 = jnp.float32
    src = edge_index[0]
    dst = edge_index[1]
    npade = NE_PAD - N_EDGES
    pad_src = jnp.asarray(np.arange(npade, dtype=np.int32) % N_NODES)
    pad_dst = jnp.asarray(
        N_NODES + (np.arange(npade, dtype=np.int32) % (NPAD - N_NODES)))
    with jax.ensure_compile_time_eval():
        eps = jax.random.normal(jax.random.key(42), (BATCH, 16), f32)
    dst_r = jnp.concatenate([dst, pad_dst]).reshape(NW, NCH, CHUNK)

    ones16 = jnp.ones((CHUNK, RSW), f32)
    zeros16 = jnp.zeros((ROWS_PER_SUB, RSW), f32)
    zeros64 = jnp.zeros((ROWS_PER_SUB, 64), f32)
    zeros32 = jnp.zeros((ROWS_PER_SUB, 32), f32)

    cnt0, cnt1 = _deg_kernel(dst_r, ones16, zeros16)
    h1 = _mmh(x, Wpe)
    src_d = lax.optimization_barrier((src, cnt0))[0]
    src_r = jnp.concatenate([src_d, pad_src]).reshape(NW, NCH, CHUNK)

    rs1 = lax.rsqrt(1.0 + cnt0[:, :1] + cnt1[:, :1])
    g1 = h1 * rs1
    t10, t11 = _agg64(src_r, dst_r, g1, zeros64)
    h0 = jnp.maximum((t10 + t11 + g1) * rs1 + bpe, 0.0)
    g2 = _mm2(h0, Wse) * rs1
    t20, t21 = _agg32(src_r, dst_r, g2, zeros32)

    pc = jnp.concatenate([node_pair[:, 0], node_pair[:, 1]]
                         ).reshape(NW, PCH, CHUNK)
    rs_tab = jnp.broadcast_to(rs1, (NPAD, RSW))
    s, grs = _pair_gather(pc, t20, t21, g2, rs_tab)

    h_pair = jnp.maximum(s * grs[:, :1] + bse, 0.0)
    hadd = h_pair[:BATCH] + h_pair[BATCH:]

    xrec, ap, dp, mean, logstd = _decoder(
        hadd, eps, Wm, bm.reshape(1, 16),
        Wls, bls.reshape(1, 16), Wsd, bsd.reshape(1, 32),
        Wpd1, bpd1.reshape(1, 64), Wpd2, bpd2.reshape(1, 128),
        Wa1, ba1.reshape(1, 8), Wa2, ba2.reshape(1, 2),
        Wd1, bd1.reshape(1, 8), Wd2, bd2.reshape(1, 2))
    return (xrec, ap, dp, mean, logstd)


# speedup vs baseline: 1.0835x; 1.0835x over previous
import functools

import jax
import jax.numpy as jnp
import numpy as np
from jax import lax
from jax.experimental import pallas as pl
from jax.experimental.pallas import tpu as pltpu
from jax.experimental.pallas import tpu_sc as plsc

N_NODES = 10000
NPAD = 10112
ROWS_PER_SUB = NPAD // 16
N_EDGES = 320000
NW = 32
CHUNK = 128
NCH = 80
NE_PAD = NW * NCH * CHUNK
RSW = 16
BATCH = 4096
NBUF = 8

_mesh = plsc.VectorSubcoreMesh(core_axis_name="c", subcore_axis_name="s")
_sc_params = pltpu.CompilerParams(use_tc_tiling_on_sc=False)


@functools.partial(
    pl.kernel,
    out_type=[
        jax.ShapeDtypeStruct((NPAD, RSW), jnp.float32),
        jax.ShapeDtypeStruct((NPAD, RSW), jnp.float32),
    ],
    mesh=_mesh,
    compiler_params=_sc_params,
    scratch_types=[
        pltpu.VMEM((NCH, CHUNK), jnp.int32),
        pltpu.VMEM((CHUNK, RSW), jnp.float32),
        pltpu.VMEM_SHARED((NPAD, RSW), jnp.float32),
        [pltpu.SemaphoreType.DMA for _ in range(8)],
    ],
)
def _deg_kernel(dst_hbm, ones_hbm, zeros_hbm, out0, out1, idx_v, ones_v, acc,
                sems):
    c = lax.axis_index("c")
    s = lax.axis_index("s")
    w = c * 16 + s
    pltpu.sync_copy(zeros_hbm, acc.at[pl.ds(s * ROWS_PER_SUB, ROWS_PER_SUB)])
    pltpu.sync_copy(ones_hbm, ones_v)
    pltpu.sync_copy(dst_hbm.at[w], idx_v)
    plsc.subcore_barrier()

    def step(j, carry):
        c0 = j * 8
        for k in range(8):
            pltpu.async_copy(ones_v, acc.at[idx_v.at[c0 + k]], sems[k], add=True)
        for k in range(8):
            pltpu.make_async_copy(ones_v, acc.at[idx_v.at[0]], sems[k]).wait()
        return carry

    lax.fori_loop(0, NCH // 8, step, 0)
    plsc.subcore_barrier()
    r0 = s * ROWS_PER_SUB
    sl = pl.ds(r0, ROWS_PER_SUB)

    @pl.when(c == 0)
    def _():
        pltpu.sync_copy(acc.at[sl], out0.at[sl])

    @pl.when(c == 1)
    def _():
        pltpu.sync_copy(acc.at[sl], out1.at[sl])


def _make_agg_kernel(feat):
    @functools.partial(
        pl.kernel,
        out_type=[
            jax.ShapeDtypeStruct((NPAD, feat), jnp.float32),
            jax.ShapeDtypeStruct((NPAD, feat), jnp.float32),
        ],
        mesh=_mesh,
        compiler_params=_sc_params,
        scratch_types=[
            pltpu.VMEM((NCH, CHUNK), jnp.int32),
            pltpu.VMEM((NCH, CHUNK), jnp.int32),
            [pltpu.VMEM((CHUNK, feat), jnp.float32) for _ in range(NBUF)],
            pltpu.VMEM_SHARED((NPAD, feat), jnp.float32),
            [pltpu.SemaphoreType.DMA for _ in range(NBUF)],
            [pltpu.SemaphoreType.DMA for _ in range(NBUF)],
        ],
    )
    def _agg(src_hbm, dst_hbm, g_hbm, zeros_hbm, out0, out1,
             srcv, dstv, rows, acc, gsem, ssem):
        c = lax.axis_index("c")
        s = lax.axis_index("s")
        w = c * 16 + s
        pltpu.sync_copy(zeros_hbm, acc.at[pl.ds(s * ROWS_PER_SUB, ROWS_PER_SUB)])
        pltpu.sync_copy(src_hbm.at[w], srcv)
        pltpu.sync_copy(dst_hbm.at[w], dstv)
        plsc.subcore_barrier()

        for k in range(NBUF):
            pltpu.async_copy(g_hbm.at[srcv.at[k]], rows[k], gsem[k])

        def step(j, carry):
            c0 = j * NBUF
            for k in range(NBUF):
                pltpu.make_async_copy(g_hbm.at[srcv.at[0]], rows[k], gsem[k]).wait()
                pltpu.async_copy(rows[k], acc.at[dstv.at[c0 + k]], ssem[k],
                                 add=True)
            for k in range(NBUF):
                pltpu.make_async_copy(rows[k], acc.at[dstv.at[0]], ssem[k]).wait()
                pltpu.async_copy(g_hbm.at[srcv.at[c0 + NBUF + k]], rows[k],
                                 gsem[k])
            return carry

        lax.fori_loop(0, NCH // NBUF - 1, step, 0)
        c0 = NCH - NBUF
        for k in range(NBUF):
            pltpu.make_async_copy(g_hbm.at[srcv.at[0]], rows[k], gsem[k]).wait()
            pltpu.async_copy(rows[k], acc.at[dstv.at[c0 + k]], ssem[k], add=True)
        for k in range(NBUF):
            pltpu.make_async_copy(rows[k], acc.at[dstv.at[0]], ssem[k]).wait()
        plsc.subcore_barrier()
        sl = pl.ds(s * ROWS_PER_SUB, ROWS_PER_SUB)

        @pl.when(c == 0)
        def _():
            pltpu.sync_copy(acc.at[sl], out0.at[sl])

        @pl.when(c == 1)
        def _():
            pltpu.sync_copy(acc.at[sl], out1.at[sl])

    return _agg


_agg64 = _make_agg_kernel(64)
_agg32 = _make_agg_kernel(32)


PBATCH = 2 * BATCH
PW = PBATCH // NW
PCH = PW // CHUNK


@functools.partial(
    pl.kernel,
    out_type=[
        jax.ShapeDtypeStruct((PBATCH, 32), jnp.float32),
        jax.ShapeDtypeStruct((PBATCH, RSW), jnp.float32),
    ],
    mesh=_mesh,
    compiler_params=_sc_params,
    scratch_types=[
        pltpu.VMEM((PCH, CHUNK), jnp.int32),
        pltpu.VMEM((CHUNK, 32), jnp.float32),
        pltpu.VMEM((CHUNK, 32), jnp.float32),
        pltpu.VMEM((CHUNK, 32), jnp.float32),
        pltpu.VMEM((CHUNK, RSW), jnp.float32),
        pltpu.SemaphoreType.DMA,
    ],
)
def _pair_gather(pc_hbm, t20_hbm, t21_hbm, g2_hbm, rs_hbm,
                 s_out, grs_out, idx2, b0, b1, b2, br, sem):
    w = _worker_id()
    pltpu.sync_copy(pc_hbm.at[w], idx2)
    for j in range(PCH):
        base = w * (PCH * CHUNK) + j * CHUNK
        pltpu.async_copy(t20_hbm.at[idx2.at[j]], b0, sem)
        pltpu.async_copy(t21_hbm.at[idx2.at[j]], b1, sem)
        pltpu.async_copy(g2_hbm.at[idx2.at[j]], b2, sem)
        pltpu.async_copy(rs_hbm.at[idx2.at[j]], br, sem)
        pltpu.make_async_copy(t20_hbm.at[idx2.at[j]], b0, sem).wait()
        pltpu.make_async_copy(t20_hbm.at[idx2.at[j]], b1, sem).wait()
        pltpu.make_async_copy(t20_hbm.at[idx2.at[j]], b2, sem).wait()
        pltpu.make_async_copy(rs_hbm.at[idx2.at[j]], br, sem).wait()

        def vsum(i, carry):
            r = i // 2
            l = (i % 2) * 16
            b0[r, pl.ds(l, 16)] = (b0[r, pl.ds(l, 16)] + b1[r, pl.ds(l, 16)]
                                   + b2[r, pl.ds(l, 16)])
            return carry

        lax.fori_loop(0, CHUNK * 2, vsum, 0)
        pltpu.sync_copy(b0, s_out.at[pl.ds(base, CHUNK)])
        pltpu.sync_copy(br, grs_out.at[pl.ds(base, CHUNK)])


def _worker_id():
    return lax.axis_index("c") * 16 + lax.axis_index("s")


_BLK = 2048
_GRID = (NPAD + _BLK - 1) // _BLK


def _mm1_body(x_ref, w_ref, cnt0_ref, cnt1_ref, g1_ref, rs_ref):
    rs = lax.rsqrt(1.0 + cnt0_ref[...] + cnt1_ref[...])
    rs_ref[...] = rs
    h = jnp.dot(x_ref[...], w_ref[...], preferred_element_type=jnp.float32)
    g1_ref[...] = h * rs[:, :1]


def _mm1(x, Wpe, cnt0, cnt1):
    return pl.pallas_call(
        _mm1_body,
        grid=(_GRID,),
        in_specs=[
            pl.BlockSpec((_BLK, 128), lambda i: (i, 0)),
            pl.BlockSpec((128, 64), lambda i: (0, 0)),
            pl.BlockSpec((_BLK, RSW), lambda i: (i, 0)),
            pl.BlockSpec((_BLK, RSW), lambda i: (i, 0)),
        ],
        out_specs=[
            pl.BlockSpec((_BLK, 64), lambda i: (i, 0)),
            pl.BlockSpec((_BLK, RSW), lambda i: (i, 0)),
        ],
        out_shape=[
            jax.ShapeDtypeStruct((NPAD, 64), jnp.float32),
            jax.ShapeDtypeStruct((NPAD, RSW), jnp.float32),
        ],
    )(x, Wpe, cnt0, cnt1)


def _mm2_body(t10_ref, t11_ref, g1_ref, rs_ref, w_ref, b_ref, g2_ref):
    rs1 = rs_ref[:, :1]
    h0 = jnp.maximum(
        (t10_ref[...] + t11_ref[...] + g1_ref[...]) * rs1 + b_ref[...], 0.0)
    g2_ref[...] = jnp.dot(h0, w_ref[...], preferred_element_type=jnp.float32) * rs1


def _mm2(t10, t11, g1, rs, Wse, bpe):
    return pl.pallas_call(
        _mm2_body,
        grid=(_GRID,),
        in_specs=[
            pl.BlockSpec((_BLK, 64), lambda i: (i, 0)),
            pl.BlockSpec((_BLK, 64), lambda i: (i, 0)),
            pl.BlockSpec((_BLK, 64), lambda i: (i, 0)),
            pl.BlockSpec((_BLK, RSW), lambda i: (i, 0)),
            pl.BlockSpec((64, 32), lambda i: (0, 0)),
            pl.BlockSpec((1, 64), lambda i: (0, 0)),
        ],
        out_specs=pl.BlockSpec((_BLK, 32), lambda i: (i, 0)),
        out_shape=jax.ShapeDtypeStruct((NPAD, 32), jnp.float32),
    )(t10, t11, g1, rs, Wse, bpe)


def _softmax2(z):
    m = jnp.max(z, axis=-1, keepdims=True)
    e = jnp.exp(z - m)
    return e / jnp.sum(e, axis=-1, keepdims=True)


def _dec_body(s_ref, grs_ref, eps_ref, bse_ref,
              wm_ref, bm_ref, wls_ref, bls_ref, wsd_ref, bsd_ref,
              wpd1_ref, bpd1_ref, wpd2_ref, bpd2_ref,
              wa1_ref, ba1_ref, wa2_ref, ba2_ref,
              wd1_ref, bd1_ref, wd2_ref, bd2_ref,
              xrec_ref, ap_ref, dp_ref, mean_ref, logstd_ref):
    h = jnp.maximum(s_ref[...] * grs_ref[:, :1] + bse_ref[...], 0.0)
    hadd = h[:BATCH] + h[BATCH:]

    def mm(a, w_ref, b_ref):
        return jnp.dot(a, w_ref[...], preferred_element_type=jnp.float32) + b_ref[...]

    mean = mm(hadd, wm_ref, bm_ref)
    logstd = mm(hadd, wls_ref, bls_ref)
    mean_ref[...] = mean
    logstd_ref[...] = logstd
    nz = mean + eps_ref[...] * jnp.exp(logstd)
    xr = jnp.maximum(mm(nz, wsd_ref, bsd_ref), 0.0)
    xr = jnp.maximum(mm(xr, wpd1_ref, bpd1_ref), 0.0)
    xrec_ref[...] = mm(xr, wpd2_ref, bpd2_ref)
    ap_ref[...] = _softmax2(mm(jnp.maximum(mm(nz, wa1_ref, ba1_ref), 0.0),
                               wa2_ref, ba2_ref))
    dp_ref[...] = _softmax2(mm(jnp.maximum(mm(nz, wd1_ref, bd1_ref), 0.0),
                               wd2_ref, bd2_ref))


def _decoder(s, grs, eps, bse, Wm, bm, Wls, bls, Wsd, bsd,
             Wpd1, bpd1, Wpd2, bpd2, Wa1, ba1, Wa2, ba2, Wd1, bd1, Wd2, bd2):
    return pl.pallas_call(
        _dec_body,
        out_shape=[
            jax.ShapeDtypeStruct((BATCH, 128), jnp.float32),
            jax.ShapeDtypeStruct((BATCH, 2), jnp.float32),
            jax.ShapeDtypeStruct((BATCH, 2), jnp.float32),
            jax.ShapeDtypeStruct((BATCH, 16), jnp.float32),
            jax.ShapeDtypeStruct((BATCH, 16), jnp.float32),
        ],
    )(s, grs, eps, bse, Wm, bm, Wls, bls, Wsd, bsd,
      Wpd1, bpd1, Wpd2, bpd2, Wa1, ba1, Wa2, ba2, Wd1, bd1, Wd2, bd2)


def kernel(x, edge_index, node_pair, rate, Wpe, bpe, Wse, bse, Wm, bm, Wls,
           bls, Wsd, bsd, Wpd1, bpd1, Wpd2, bpd2, Wa1, ba1, Wa2, ba2, Wd1,
           bd1, Wd2, bd2):
    f32 = jnp.float32
    src = edge_index[0]
    dst = edge_index[1]
    npade = NE_PAD - N_EDGES
    pad_src = jnp.asarray(np.arange(npade, dtype=np.int32) % N_NODES)
    pad_dst = jnp.asarray(
        N_NODES + (np.arange(npade, dtype=np.int32) % (NPAD - N_NODES)))
    with jax.ensure_compile_time_eval():
        eps = jax.random.normal(jax.random.key(42), (BATCH, 16), f32)
    dst_r = jnp.concatenate([dst, pad_dst]).reshape(NW, NCH, CHUNK)

    ones16 = jnp.ones((CHUNK, RSW), f32)
    zeros16 = jnp.zeros((ROWS_PER_SUB, RSW), f32)
    zeros64 = jnp.zeros((ROWS_PER_SUB, 64), f32)
    zeros32 = jnp.zeros((ROWS_PER_SUB, 32), f32)

    cnt0, cnt1 = _deg_kernel(dst_r, ones16, zeros16)
    src_d = lax.optimization_barrier((src, cnt0))[0]
    src_r = jnp.concatenate([src_d, pad_src]).reshape(NW, NCH, CHUNK)

    g1, rs = _mm1(x, Wpe, cnt0, cnt1)
    t10, t11 = _agg64(src_r, dst_r, g1, zeros64)
    g2 = _mm2(t10, t11, g1, rs, Wse, bpe.reshape(1, 64))
    t20, t21 = _agg32(src_r, dst_r, g2, zeros32)

    pc = jnp.concatenate([node_pair[:, 0], node_pair[:, 1]]
                         ).reshape(NW, PCH, CHUNK)
    s, grs = _pair_gather(pc, t20, t21, g2, rs)

    xrec, ap, dp, mean, logstd = _decoder(
        s, grs, eps, bse.reshape(1, 32), Wm, bm.reshape(1, 16),
        Wls, bls.reshape(1, 16), Wsd, bsd.reshape(1, 32),
        Wpd1, bpd1.reshape(1, 64), Wpd2, bpd2.reshape(1, 128),
        Wa1, ba1.reshape(1, 8), Wa2, ba2.reshape(1, 2),
        Wd1, bd1.reshape(1, 8), Wd2, bd2.reshape(1, 2))
    return (xrec, ap, dp, mean, logstd)
